# baseline (speedup 1.0000x reference)
import functools

import jax
import jax.numpy as jnp
from jax import lax
from jax.experimental import pallas as pl
from jax.experimental.pallas import tpu as pltpu
from jax.experimental.pallas import tpu_sc as plsc

B = 16384
H = 20
N = 5
D = 64

NC = 2
NS = 16
NW = NC * NS

CH = 512


def _sc_gather_body(emb_hbm, delta_hbm, xs_hbm, ys_hbm, hs_hbm, ns_hbm,
                    x_out, y_out, h_out, n_out, d_out,
                    idx_v, rows_v, dv, sem):
    wid = lax.axis_index("s") * NC + lax.axis_index("c")

    def run_job(idx_hbm, out_hbm, rows_per_worker):
        nch = rows_per_worker // CH
        base = wid * rows_per_worker

        def body(i, _):
            off = base + i * CH
            pltpu.sync_copy(idx_hbm.at[pl.ds(off, CH)], idx_v)
            pltpu.async_copy(emb_hbm.at[idx_v], rows_v, sem).wait()
            pltpu.sync_copy(rows_v, out_hbm.at[pl.ds(off, CH)])
            return 0

        lax.fori_loop(0, nch, body, 0)

    run_job(xs_hbm, x_out, B // NW)
    run_job(ys_hbm, y_out, B // NW)
    run_job(hs_hbm, h_out, B * H // NW)
    run_job(ns_hbm, n_out, B * N // NW)

    base = wid * (B // NW)
    pltpu.sync_copy(xs_hbm.at[pl.ds(base, B // NW)], idx_v)
    pltpu.async_copy(delta_hbm.at[idx_v], dv, sem).wait()
    pltpu.sync_copy(dv, d_out.at[pl.ds(base, B // NW)])


def _sc_gather(emb, delta_flat, xs, ys, hs_flat, ns_flat):
    mesh = plsc.VectorSubcoreMesh(core_axis_name="c", subcore_axis_name="s")
    f = pl.kernel(
        _sc_gather_body,
        out_type=[
            jax.ShapeDtypeStruct((B, D), jnp.float32),
            jax.ShapeDtypeStruct((B, D), jnp.float32),
            jax.ShapeDtypeStruct((B * H, D), jnp.float32),
            jax.ShapeDtypeStruct((B * N, D), jnp.float32),
            jax.ShapeDtypeStruct((B,), jnp.float32),
        ],
        mesh=mesh,
        compiler_params=pltpu.CompilerParams(use_tc_tiling_on_sc=False),
        scratch_types=[
            pltpu.VMEM((CH,), jnp.int32),
            pltpu.VMEM((CH, D), jnp.float32),
            pltpu.VMEM((CH,), jnp.float32),
            pltpu.SemaphoreType.DMA,
        ],
    )
    return f(emb, delta_flat, xs, ys, hs_flat, ns_flat)


def _log_sigmoid(z):
    return jnp.minimum(z, 0.0) - jnp.log1p(jnp.exp(-jnp.abs(z)))


def _tc_body(x_ref, y_ref, h_ref, n_ref, delta_ref, et_ref, ht_ref, mask_ref,
             loss_ref):
    BB = x_ref.shape[0]
    x = x_ref[...]
    y = y_ref[...]
    delta = delta_ref[...]
    et = et_ref[...]
    ht = ht_ref[...]
    mask = mask_ref[...]

    d_time = jnp.abs(et - ht)
    xx = jnp.sum(x * x, axis=-1, keepdims=True)
    dxy = x - y
    p_mu = -jnp.sum(dxy * dxy, axis=-1, keepdims=True)

    alpha_cols = []
    hn2_cols = []
    for j in range(H):
        hj = h_ref[j]
        hn2_cols.append(jnp.sum(hj * hj, axis=-1, keepdims=True))
        xh = jnp.sum(x * hj, axis=-1, keepdims=True)
        alpha_cols.append(2.0 * xh - xx - hn2_cols[-1])
    alpha = jnp.concatenate(alpha_cols, axis=1)
    hn2 = jnp.concatenate(hn2_cols, axis=1)

    amax = jnp.max(alpha, axis=1, keepdims=True)
    ea = jnp.exp(alpha - amax)
    attn = ea / jnp.sum(ea, axis=1, keepdims=True)
    decay = jnp.exp(delta * d_time) * mask
    w = attn * decay
    p_lambda = p_mu[:, 0] + jnp.sum(w * alpha, axis=-1)

    W = jnp.sum(w, axis=1, keepdims=True)
    S2 = jnp.sum(w * hn2, axis=1, keepdims=True)
    hbar = jnp.zeros((BB, D), jnp.float32)
    for j in range(H):
        hbar = hbar + w[:, j:j + 1] * h_ref[j]

    loss = _log_sigmoid(p_lambda)
    for k in range(N):
        nk = n_ref[k]
        nn2 = jnp.sum(nk * nk, axis=-1, keepdims=True)
        xn = jnp.sum(x * nk, axis=-1, keepdims=True)
        hdotn = jnp.sum(hbar * nk, axis=-1, keepdims=True)
        n_lambda = (2.0 * xn - xx - nn2 - S2 - W * nn2
                    + 2.0 * hdotn)
        loss = loss - _log_sigmoid(n_lambda[:, 0])
    loss_ref[...] = loss[:, None]


def _tc_compute(x_e, y_e, hT, nT, delta, e_times, h_times, mask):
    BB = 1024
    grid = (B // BB,)
    out = pl.pallas_call(
        _tc_body,
        grid=grid,
        compiler_params=pltpu.CompilerParams(
            vmem_limit_bytes=100 * 1024 * 1024),
        in_specs=[
            pl.BlockSpec((BB, D), lambda i: (i, 0)),
            pl.BlockSpec((BB, D), lambda i: (i, 0)),
            pl.BlockSpec((H, BB, D), lambda i: (0, i, 0)),
            pl.BlockSpec((N, BB, D), lambda i: (0, i, 0)),
            pl.BlockSpec((BB, 1), lambda i: (i, 0)),
            pl.BlockSpec((BB, 1), lambda i: (i, 0)),
            pl.BlockSpec((BB, H), lambda i: (i, 0)),
            pl.BlockSpec((BB, H), lambda i: (i, 0)),
        ],
        out_specs=pl.BlockSpec((BB, 1), lambda i: (i, 0)),
        out_shape=jax.ShapeDtypeStruct((B, 1), jnp.float32),
    )(x_e, y_e, hT, nT, delta, e_times, h_times, mask)
    return out[:, 0]


def kernel(xs, ys, e_times, hs, h_times, neg_node, h_times_mask, emb_table,
           delta_table):
    xs = xs.astype(jnp.int32)
    ys = ys.astype(jnp.int32)
    hsT_flat = hs.astype(jnp.int32).T.reshape(-1)
    nsT_flat = neg_node.astype(jnp.int32).T.reshape(-1)
    delta_flat = delta_table.reshape(-1)

    x_e, y_e, h_e, n_e, delta = _sc_gather(emb_table, delta_flat, xs, ys,
                                           hsT_flat, nsT_flat)
    return _tc_compute(x_e, y_e, h_e.reshape(H, B, D), n_e.reshape(N, B, D),
                       delta[:, None], e_times[:, None], h_times,
                       h_times_mask)


# speedup vs baseline: 1.0334x; 1.0334x over previous
import functools

import jax
import jax.numpy as jnp
from jax import lax
from jax.experimental import pallas as pl
from jax.experimental.pallas import tpu as pltpu
from jax.experimental.pallas import tpu_sc as plsc

B = 16384
H = 20
N = 5
D = 64

NC = 2
NS = 16
NW = NC * NS

CH = 512


def _sc_gather_body(emb_hbm, delta_hbm, xs_hbm, ys_hbm, hs_hbm, ns_hbm,
                    x_out, y_out, h_out, n_out, d_out,
                    idx_v, rows_v, dv, sem_g, sem_s):
    wid = lax.axis_index("s") * NC + lax.axis_index("c")

    def run_job(idx_hbm, out_hbm, rows_per_worker):
        nch = rows_per_worker // CH
        base = wid * rows_per_worker
        pltpu.sync_copy(idx_hbm.at[pl.ds(base, rows_per_worker)],
                        idx_v.at[pl.ds(0, rows_per_worker)])

        def gather(i, p):
            return pltpu.make_async_copy(
                emb_hbm.at[idx_v.at[pl.ds(i * CH, CH)]],
                rows_v.at[pl.ds(p * CH, CH)], sem_g)

        def store(i, p):
            return pltpu.make_async_copy(
                rows_v.at[pl.ds(p * CH, CH)],
                out_hbm.at[pl.ds(base + i * CH, CH)], sem_s)

        gather(0, 0).start()

        def body(i, _):
            p = lax.rem(i, 2)
            q = 1 - p

            @pl.when(i >= 1)
            def _():
                store(i - 1, q).wait()

            @pl.when(i + 1 < nch)
            def _():
                gather(i + 1, q).start()

            gather(i, p).wait()
            store(i, p).start()
            return 0

        lax.fori_loop(0, nch, body, 0)
        store(nch - 1, (nch - 1) % 2).wait()

    run_job(xs_hbm, x_out, B // NW)
    run_job(ys_hbm, y_out, B // NW)
    run_job(hs_hbm, h_out, B * H // NW)
    run_job(ns_hbm, n_out, B * N // NW)

    base = wid * (B // NW)
    pltpu.sync_copy(xs_hbm.at[pl.ds(base, B // NW)],
                    idx_v.at[pl.ds(0, B // NW)])
    pltpu.async_copy(delta_hbm.at[idx_v.at[pl.ds(0, B // NW)]], dv,
                     sem_g).wait()
    pltpu.sync_copy(dv, d_out.at[pl.ds(base, B // NW)])


def _sc_gather(emb, delta_flat, xs, ys, hs_flat, ns_flat):
    mesh = plsc.VectorSubcoreMesh(core_axis_name="c", subcore_axis_name="s")
    f = pl.kernel(
        _sc_gather_body,
        out_type=[
            jax.ShapeDtypeStruct((B, D), jnp.float32),
            jax.ShapeDtypeStruct((B, D), jnp.float32),
            jax.ShapeDtypeStruct((B * H, D), jnp.float32),
            jax.ShapeDtypeStruct((B * N, D), jnp.float32),
            jax.ShapeDtypeStruct((B,), jnp.float32),
        ],
        mesh=mesh,
        compiler_params=pltpu.CompilerParams(use_tc_tiling_on_sc=False),
        scratch_types=[
            pltpu.VMEM((B * H // NW,), jnp.int32),
            pltpu.VMEM((2 * CH, D), jnp.float32),
            pltpu.VMEM((B // NW,), jnp.float32),
            pltpu.SemaphoreType.DMA,
            pltpu.SemaphoreType.DMA,
        ],
    )
    return f(emb, delta_flat, xs, ys, hs_flat, ns_flat)


def _log_sigmoid(z):
    return jnp.minimum(z, 0.0) - jnp.log1p(jnp.exp(-jnp.abs(z)))


def _tc_body(x_ref, y_ref, h_ref, n_ref, delta_ref, et_ref, ht_ref, mask_ref,
             loss_ref):
    BB = x_ref.shape[0]
    x = x_ref[...]
    y = y_ref[...]
    delta = delta_ref[...]
    et = et_ref[...]
    ht = ht_ref[...]
    mask = mask_ref[...]

    d_time = jnp.abs(et - ht)
    xx = jnp.sum(x * x, axis=-1, keepdims=True)
    dxy = x - y
    p_mu = -jnp.sum(dxy * dxy, axis=-1, keepdims=True)

    alpha_cols = []
    hn2_cols = []
    for j in range(H):
        hj = h_ref[j]
        hn2_cols.append(jnp.sum(hj * hj, axis=-1, keepdims=True))
        xh = jnp.sum(x * hj, axis=-1, keepdims=True)
        alpha_cols.append(2.0 * xh - xx - hn2_cols[-1])
    alpha = jnp.concatenate(alpha_cols, axis=1)
    hn2 = jnp.concatenate(hn2_cols, axis=1)

    amax = jnp.max(alpha, axis=1, keepdims=True)
    ea = jnp.exp(alpha - amax)
    attn = ea / jnp.sum(ea, axis=1, keepdims=True)
    decay = jnp.exp(delta * d_time) * mask
    w = attn * decay
    p_lambda = p_mu[:, 0] + jnp.sum(w * alpha, axis=-1)

    W = jnp.sum(w, axis=1, keepdims=True)
    S2 = jnp.sum(w * hn2, axis=1, keepdims=True)
    hbar = jnp.zeros((BB, D), jnp.float32)
    for j in range(H):
        hbar = hbar + w[:, j:j + 1] * h_ref[j]

    loss = _log_sigmoid(p_lambda)
    for k in range(N):
        nk = n_ref[k]
        nn2 = jnp.sum(nk * nk, axis=-1, keepdims=True)
        xn = jnp.sum(x * nk, axis=-1, keepdims=True)
        hdotn = jnp.sum(hbar * nk, axis=-1, keepdims=True)
        n_lambda = (2.0 * xn - xx - nn2 - S2 - W * nn2
                    + 2.0 * hdotn)
        loss = loss - _log_sigmoid(n_lambda[:, 0])
    loss_ref[...] = loss[:, None]


def _tc_compute(x_e, y_e, hT, nT, delta, e_times, h_times, mask):
    BB = 512
    grid = (B // BB,)
    out = pl.pallas_call(
        _tc_body,
        grid=grid,
        compiler_params=pltpu.CompilerParams(
            vmem_limit_bytes=100 * 1024 * 1024),
        in_specs=[
            pl.BlockSpec((BB, D), lambda i: (i, 0)),
            pl.BlockSpec((BB, D), lambda i: (i, 0)),
            pl.BlockSpec((H, BB, D), lambda i: (0, i, 0)),
            pl.BlockSpec((N, BB, D), lambda i: (0, i, 0)),
            pl.BlockSpec((BB, 1), lambda i: (i, 0)),
            pl.BlockSpec((BB, 1), lambda i: (i, 0)),
            pl.BlockSpec((BB, H), lambda i: (i, 0)),
            pl.BlockSpec((BB, H), lambda i: (i, 0)),
        ],
        out_specs=pl.BlockSpec((BB, 1), lambda i: (i, 0)),
        out_shape=jax.ShapeDtypeStruct((B, 1), jnp.float32),
    )(x_e, y_e, hT, nT, delta, e_times, h_times, mask)
    return out[:, 0]


def kernel(xs, ys, e_times, hs, h_times, neg_node, h_times_mask, emb_table,
           delta_table):
    xs = xs.astype(jnp.int32)
    ys = ys.astype(jnp.int32)
    hsT_flat = hs.astype(jnp.int32).T.reshape(-1)
    nsT_flat = neg_node.astype(jnp.int32).T.reshape(-1)
    delta_flat = delta_table.reshape(-1)

    x_e, y_e, h_e, n_e, delta = _sc_gather(emb_table, delta_flat, xs, ys,
                                           hsT_flat, nsT_flat)
    return _tc_compute(x_e, y_e, h_e.reshape(H, B, D), n_e.reshape(N, B, D),
                       delta[:, None], e_times[:, None], h_times,
                       h_times_mask)
